# baseline (speedup 1.0000x reference)
import functools

import jax
import jax.numpy as jnp
from jax import lax
from jax.experimental import pallas as pl
from jax.experimental.pallas import tpu as pltpu
from jax.experimental.pallas import tpu_sc as plsc

B_TOTAL = 4 * 256 * 256
D = 512
V = 64
NC = 2
NS = 16
NW = NC * NS
BPW = B_TOTAL // NW
K = 64
NCHUNK = BPW // K
NPAIR = NCHUNK // 2


def _sc_lookup(g_flat, gt_flat, table_rep):
    mesh = plsc.VectorSubcoreMesh(core_axis_name="c", subcore_axis_name="s")

    @functools.partial(
        pl.kernel,
        mesh=mesh,
        out_type=jax.ShapeDtypeStruct((B_TOTAL, D), jnp.float32),
        scratch_types=[
            pltpu.VMEM((BPW,), jnp.int32),
            pltpu.VMEM((BPW,), jnp.int32),
            pltpu.VMEM((2, K, D), jnp.float32),
            pltpu.SemaphoreType.DMA,
            pltpu.SemaphoreType.DMA,
            pltpu.SemaphoreType.DMA,
            pltpu.SemaphoreType.DMA,
        ],
    )
    def body(g_hbm, gt_hbm, table_hbm, out_hbm, idx_v, add_v, rows_v,
             gsem0, gsem1, osem0, osem1):
        wid = lax.axis_index("s") * NC + lax.axis_index("c")
        base = wid * BPW
        toff = wid * V

        pltpu.sync_copy(g_hbm.at[pl.ds(base, BPW)], idx_v)
        pltpu.sync_copy(gt_hbm.at[pl.ds(base, BPW)], add_v)

        def add_chunk(i, carry):
            sl = pl.ds(i * 16, 16)
            idx_v[sl] = idx_v[sl] + add_v[sl] + toff
            return carry

        lax.fori_loop(0, BPW // 16, add_chunk, 0)

        def start_gather(c, slot, sem):
            pltpu.async_copy(
                table_hbm.at[idx_v.at[pl.ds(c * K, K)]], rows_v.at[slot], sem)

        def wait_gather(slot, sem):
            pltpu.make_async_copy(
                table_hbm.at[pl.ds(0, K)], rows_v.at[slot], sem).wait()

        def start_out(c, slot, sem):
            pltpu.async_copy(
                rows_v.at[slot], out_hbm.at[pl.ds(base + c * K, K)], sem)

        def wait_out(slot, sem):
            pltpu.make_async_copy(
                rows_v.at[slot], out_hbm.at[pl.ds(base, K)], sem).wait()

        start_gather(0, 0, gsem0)

        def pair(p, carry):
            a = 2 * p
            b = a + 1
            wait_gather(0, gsem0)

            @pl.when(p > 0)
            def _():
                wait_out(1, osem1)

            start_gather(b, 1, gsem1)
            start_out(a, 0, osem0)
            wait_gather(1, gsem1)
            wait_out(0, osem0)

            @pl.when(p < NPAIR - 1)
            def _():
                start_gather(a + 2, 0, gsem0)

            start_out(b, 1, osem1)
            return carry

        lax.fori_loop(0, NPAIR, pair, 0)
        wait_out(1, osem1)

    return body(g_flat, gt_flat, table_rep)


def kernel(graphs, spec_type, normal_type):
    table = jnp.concatenate((spec_type, normal_type), axis=0)
    table_rep = jnp.tile(table, (NW, 1))
    g_flat = graphs.reshape(B_TOTAL)
    gt_flat = jnp.transpose(graphs, (0, 2, 1)).reshape(B_TOTAL)
    out = _sc_lookup(g_flat, gt_flat, table_rep)
    return out.reshape(4, 256, 256, D)
